# Initial kernel scaffold; baseline (speedup 1.0000x reference)
import jax
import jax.numpy as jnp
from jax.experimental import pallas as pl


def kernel(x, edge_index, W1, a_src1, a_dst1, b1, W2, a_src2, a_dst2, b2):
    raise NotImplementedError("write your pallas kernel here")


# speedup vs baseline: 52.8097x; 52.8097x over previous
import functools

import numpy as np
import jax
import jax.numpy as jnp
from jax import lax
from jax.experimental import pallas as pl
from jax.experimental.pallas import tpu as pltpu
from jax.experimental.pallas import tpu_sc as plsc

_N = 10000
_F = 128
_NC, _NS = 2, 16
_NW = _NC * _NS
_K = 128
_E_TOT = 320000 + _N
_NCHUNK = -(-_E_TOT // (_NW * _K))
_E_PAD = _NW * _NCHUNK * _K
_RPT = 632
_NPAD = _NS * _RPT

_f32 = jnp.float32
_i32 = jnp.int32


def _make_edge_kernel(H, C):
    HC = H * C
    WS = HC + 16
    mesh = plsc.VectorSubcoreMesh(core_axis_name="c", subcore_axis_name="s")

    def body(tsrc, tdst, sidx_h, didx_h, zeros_h, out_h,
             sidx, didx, rows, arows, acc, sem_g, sem_a):
        cid = lax.axis_index("c")
        sid = lax.axis_index("s")
        g = cid * _NS + sid
        iota = lax.iota(_i32, 16)

        pltpu.sync_copy(zeros_h.at[pl.ds(sid * _RPT, _RPT)],
                        acc.at[pl.ds(sid * _RPT, _RPT)])
        plsc.subcore_barrier()

        def chunk_body(j, carry):
            pltpu.sync_copy(sidx_h.at[g, j], sidx)
            pltpu.sync_copy(didx_h.at[g, j], didx)
            cp1 = pltpu.async_copy(tsrc.at[sidx], rows, sem_g)
            cp2 = pltpu.async_copy(tdst.at[didx], arows, sem_a)
            cp1.wait()
            cp2.wait()

            def e_body(e, c):
                av = rows[e, pl.ds(HC, 16)]
                bv = arows[e, pl.ds(0, 16)]
                a = av + bv
                a = jnp.maximum(a, 0.2 * a)
                ex = jnp.exp(a)
                ex = jnp.where(iota < H, ex, 0.0)
                rows[e, pl.ds(HC, 16)] = ex
                for h in range(H):
                    scv = ex.at[iota * 0 + h].get(mode="promise_in_bounds")
                    off = h * C
                    rows[e, pl.ds(off, 16)] = rows[e, pl.ds(off, 16)] * scv
                return c
            lax.fori_loop(0, _K, e_body, 0)

            pltpu.sync_copy(rows, acc.at[didx], add=True)
            return carry

        lax.fori_loop(0, _NCHUNK, chunk_body, 0)

        plsc.subcore_barrier()
        pltpu.sync_copy(acc.at[pl.ds(sid * _RPT, _RPT)],
                        out_h.at[cid, pl.ds(sid * _RPT, _RPT)])

    return pl.kernel(
        body,
        out_type=jax.ShapeDtypeStruct((_NC, _NPAD, WS), _f32),
        mesh=mesh,
        scratch_types=[
            pltpu.VMEM((_K,), _i32),
            pltpu.VMEM((_K,), _i32),
            pltpu.VMEM((_K, WS), _f32),
            pltpu.VMEM((_K, 16), _f32),
            pltpu.VMEM_SHARED((_NPAD, WS), _f32),
            pltpu.SemaphoreType.DMA,
            pltpu.SemaphoreType.DMA,
        ],
        compiler_params=pltpu.CompilerParams(use_tc_tiling_on_sc=False),
    )


_edge1 = _make_edge_kernel(8, 16)
_edge2 = _make_edge_kernel(1, 16)


_BM = 1000


def _mm_body(x_ref, w_ref, o_ref):
    o_ref[...] = jnp.dot(x_ref[...], w_ref[...], preferred_element_type=_f32)


def _tc_matmul(x, w):
    m, k = x.shape
    n = w.shape[1]
    return pl.pallas_call(
        _mm_body,
        grid=(m // _BM,),
        in_specs=[pl.BlockSpec((_BM, k), lambda i: (i, 0)),
                  pl.BlockSpec((k, n), lambda i: (0, 0))],
        out_specs=pl.BlockSpec((_BM, n), lambda i: (i, 0)),
        out_shape=jax.ShapeDtypeStruct((m, n), _f32),
    )(x, w)


def _mid_body(acc_ref, b_ref, w_ref, r_ref, o_ref):
    s = acc_ref[0] + acc_ref[1]
    num = s[:, :128]
    den = jnp.dot(s[:, 128:136], r_ref[...], preferred_element_type=_f32)
    gv = num / den + b_ref[...]
    gv = jnp.where(gv > 0, gv, jnp.exp(gv) - 1.0)
    o_ref[...] = jnp.dot(gv, w_ref[...], preferred_element_type=_f32)


def _tc_mid(acc1, b1, w2ext, r8):
    return pl.pallas_call(
        _mid_body,
        grid=(_N // _BM,),
        in_specs=[pl.BlockSpec((_NC, _BM, 144), lambda i: (0, i, 0)),
                  pl.BlockSpec((1, 128), lambda i: (0, 0)),
                  pl.BlockSpec((128, 32), lambda i: (0, 0)),
                  pl.BlockSpec((8, 128), lambda i: (0, 0))],
        out_specs=pl.BlockSpec((_BM, 32), lambda i: (i, 0)),
        out_shape=jax.ShapeDtypeStruct((_N, 32), _f32),
    )(acc1, b1, w2ext, r8)


def _fin_body(acc_ref, b_ref, o_ref):
    s = acc_ref[0] + acc_ref[1]
    num = s[:, :16]
    den = jnp.broadcast_to(s[:, 16:17], (_BM, 16))
    o = num / den + b_ref[...]
    o_ref[...] = jnp.where(o > 0, o, jnp.exp(o) - 1.0)


def _tc_fin(acc2, b2):
    return pl.pallas_call(
        _fin_body,
        grid=(_N // _BM,),
        in_specs=[pl.BlockSpec((_NC, _BM, 32), lambda i: (0, i, 0)),
                  pl.BlockSpec((1, 16), lambda i: (0, 0))],
        out_specs=pl.BlockSpec((_BM, 16), lambda i: (i, 0)),
        out_shape=jax.ShapeDtypeStruct((_N, 16), _f32),
    )(acc2, b2)


def _build_wext(W, a_s, a_d, H, C, width):
    HC = H * C
    rows_idx = jnp.arange(HC)
    A_s = jnp.zeros((HC, H), _f32).at[rows_idx, rows_idx // C].set(a_s.reshape(-1))
    A_d = jnp.zeros((HC, H), _f32).at[rows_idx, rows_idx // C].set(a_d.reshape(-1))
    wext = jnp.concatenate([W, W @ A_s, W @ A_d], axis=1)
    return jnp.pad(wext, ((0, 0), (0, width - wext.shape[1])))


def kernel(x, edge_index, W1, a_src1, a_dst1, b1, W2, a_src2, a_dst2, b2):
    w1ext = _build_wext(W1, a_src1, a_dst1, 8, 16, 144)
    w2ext = _build_wext(W2, a_src2, a_dst2, 1, 16, 32)
    r8 = jnp.kron(jnp.eye(8, dtype=_f32), jnp.ones((1, 16), _f32))

    loops = jnp.arange(_N, dtype=_i32)
    src = jnp.concatenate([edge_index[0].astype(_i32), loops])
    dst = jnp.concatenate([edge_index[1].astype(_i32), loops])
    pad_n = _E_PAD - _E_TOT
    src = jnp.pad(src, (0, pad_n), constant_values=_N).reshape(_NW, _NCHUNK, _K)
    dst = jnp.pad(dst, (0, pad_n), constant_values=_N).reshape(_NW, _NCHUNK, _K)

    z144 = jnp.zeros((_NPAD, 144), _f32)
    z32 = jnp.zeros((_NPAD, 32), _f32)

    t1 = _tc_matmul(x, w1ext)
    t1p = jnp.pad(t1, ((0, _NPAD - _N), (0, 0)))
    d1 = jnp.pad(t1[:, 136:144], ((0, _NPAD - _N), (0, 8)))
    acc1 = _edge1(t1p, d1, src, dst, z144)

    t2 = _tc_mid(acc1, b1.reshape(1, 128), w2ext, r8)
    t2p = jnp.pad(t2, ((0, _NPAD - _N), (0, 0)))
    d2 = jnp.pad(t2[:, 17:18], ((0, _NPAD - _N), (0, 15)))
    acc2 = _edge2(t2p, d2, src, dst, z32)

    return _tc_fin(acc2, b2.reshape(1, 16))


# speedup vs baseline: 99.7928x; 1.8897x over previous
import functools

import numpy as np
import jax
import jax.numpy as jnp
from jax import lax
from jax.experimental import pallas as pl
from jax.experimental.pallas import tpu as pltpu
from jax.experimental.pallas import tpu_sc as plsc

_N = 10000
_F = 128
_NC, _NS = 2, 16
_NW = _NC * _NS
_K = 96
_E_TOT = 320000 + _N
_NCHUNK = 108
_E_PAD = _NW * _NCHUNK * _K
_RPT = 632
_NPAD = _NS * _RPT

_f32 = jnp.float32
_i32 = jnp.int32


def _make_edge_kernel(H, C):
    HC = H * C
    WS = HC + 16
    mesh = plsc.VectorSubcoreMesh(core_axis_name="c", subcore_axis_name="s")

    def body(tsrc, tdst, eidx_h, zeros_h, out_h,
             cidx0, cidx1, rows0, rows1, arows0, arows1, acc,
             sem_i0, sem_i1, sem_s0, sem_s1, sem_a0, sem_a1):
        cid = lax.axis_index("c")
        sid = lax.axis_index("s")
        g = cid * _NS + sid
        iota = lax.iota(_i32, 16)

        pltpu.sync_copy(zeros_h.at[pl.ds(sid * _RPT, _RPT)],
                        acc.at[pl.ds(sid * _RPT, _RPT)])
        plsc.subcore_barrier()

        bufs = ((cidx0, rows0, arows0, sem_i0, sem_s0, sem_a0),
                (cidx1, rows1, arows1, sem_i1, sem_s1, sem_a1))

        def start_idx(j, b):
            cidx, _, _, si, _, _ = bufs[b]
            pltpu.async_copy(eidx_h.at[g, j], cidx, si)

        def wait_idx(j, b):
            cidx, _, _, si, _, _ = bufs[b]
            pltpu.make_async_copy(eidx_h.at[g, j], cidx, si).wait()

        def start_gather(b):
            cidx, rows, arows, _, ss, sa = bufs[b]
            pltpu.async_copy(tsrc.at[cidx.at[0]], rows, ss)
            pltpu.async_copy(tdst.at[cidx.at[1]], arows, sa)

        def wait_gather(b):
            cidx, rows, arows, _, ss, sa = bufs[b]
            pltpu.make_async_copy(tsrc.at[cidx.at[0]], rows, ss).wait()
            pltpu.make_async_copy(tdst.at[cidx.at[1]], arows, sa).wait()

        def compute_scatter(b):
            cidx, rows, arows, _, _, _ = bufs[b]

            @plsc.parallel_loop(0, _K, 1, unroll=4)
            def e_body(e):
                av = rows[e, pl.ds(HC, 16)]
                bv = arows[e, pl.ds(0, 16)]
                a = av + bv
                a = jnp.maximum(a, 0.2 * a)
                ex = jnp.exp(a)
                ex = jnp.where(iota < H, ex, 0.0)
                rows[e, pl.ds(HC, 16)] = ex
                for h in range(H):
                    scv = ex.at[iota * 0 + h].get(mode="promise_in_bounds")
                    off = h * C
                    rows[e, pl.ds(off, 16)] = rows[e, pl.ds(off, 16)] * scv

            pltpu.sync_copy(rows, acc.at[cidx.at[1]], add=True)

        pltpu.sync_copy(eidx_h.at[g, 0], cidx0)
        start_gather(0)
        start_idx(1, 1)

        def step(j, b):
            jn = jnp.minimum(j + 1, _NCHUNK - 1)
            wait_idx(jn, 1 - b)
            start_gather(1 - b)
            wait_gather(b)
            compute_scatter(b)
            start_idx(jnp.minimum(j + 2, _NCHUNK - 1), b)

        def loop_body(jj, c):
            step(jj * 2, 0)
            step(jj * 2 + 1, 1)
            return c

        lax.fori_loop(0, _NCHUNK // 2, loop_body, 0)
        wait_gather(0)
        wait_idx(_NCHUNK - 1, 1)

        plsc.subcore_barrier()
        pltpu.sync_copy(acc.at[pl.ds(sid * _RPT, _RPT)],
                        out_h.at[cid, pl.ds(sid * _RPT, _RPT)])

    return pl.kernel(
        body,
        out_type=jax.ShapeDtypeStruct((_NC, _NPAD, WS), _f32),
        mesh=mesh,
        scratch_types=[
            pltpu.VMEM((2, _K), _i32),
            pltpu.VMEM((2, _K), _i32),
            pltpu.VMEM((_K, WS), _f32),
            pltpu.VMEM((_K, WS), _f32),
            pltpu.VMEM((_K, 16), _f32),
            pltpu.VMEM((_K, 16), _f32),
            pltpu.VMEM_SHARED((_NPAD, WS), _f32),
            pltpu.SemaphoreType.DMA,
            pltpu.SemaphoreType.DMA,
            pltpu.SemaphoreType.DMA,
            pltpu.SemaphoreType.DMA,
            pltpu.SemaphoreType.DMA,
            pltpu.SemaphoreType.DMA,
        ],
        compiler_params=pltpu.CompilerParams(use_tc_tiling_on_sc=False),
    )


_edge1 = _make_edge_kernel(8, 16)
_edge2 = _make_edge_kernel(1, 16)


_BM = 1000


def _mm_body(x_ref, w_ref, o_ref):
    o_ref[...] = jnp.dot(x_ref[...], w_ref[...], preferred_element_type=_f32)


def _tc_matmul(x, w):
    m, k = x.shape
    n = w.shape[1]
    return pl.pallas_call(
        _mm_body,
        grid=(m // _BM,),
        in_specs=[pl.BlockSpec((_BM, k), lambda i: (i, 0)),
                  pl.BlockSpec((k, n), lambda i: (0, 0))],
        out_specs=pl.BlockSpec((_BM, n), lambda i: (i, 0)),
        out_shape=jax.ShapeDtypeStruct((m, n), _f32),
    )(x, w)


def _mid_body(acc_ref, b_ref, w_ref, r_ref, o_ref):
    s = acc_ref[0] + acc_ref[1]
    num = s[:, :128]
    den = jnp.dot(s[:, 128:136], r_ref[...], preferred_element_type=_f32)
    gv = num / den + b_ref[...]
    gv = jnp.where(gv > 0, gv, jnp.exp(gv) - 1.0)
    o_ref[...] = jnp.dot(gv, w_ref[...], preferred_element_type=_f32)


def _tc_mid(acc1, b1, w2ext, r8):
    return pl.pallas_call(
        _mid_body,
        grid=(_N // _BM,),
        in_specs=[pl.BlockSpec((_NC, _BM, 144), lambda i: (0, i, 0)),
                  pl.BlockSpec((1, 128), lambda i: (0, 0)),
                  pl.BlockSpec((128, 32), lambda i: (0, 0)),
                  pl.BlockSpec((8, 128), lambda i: (0, 0))],
        out_specs=pl.BlockSpec((_BM, 32), lambda i: (i, 0)),
        out_shape=jax.ShapeDtypeStruct((_N, 32), _f32),
    )(acc1, b1, w2ext, r8)


def _fin_body(acc_ref, b_ref, o_ref):
    s = acc_ref[0] + acc_ref[1]
    num = s[:, :16]
    den = jnp.broadcast_to(s[:, 16:17], (_BM, 16))
    o = num / den + b_ref[...]
    o_ref[...] = jnp.where(o > 0, o, jnp.exp(o) - 1.0)


def _tc_fin(acc2, b2):
    return pl.pallas_call(
        _fin_body,
        grid=(_N // _BM,),
        in_specs=[pl.BlockSpec((_NC, _BM, 32), lambda i: (0, i, 0)),
                  pl.BlockSpec((1, 16), lambda i: (0, 0))],
        out_specs=pl.BlockSpec((_BM, 16), lambda i: (i, 0)),
        out_shape=jax.ShapeDtypeStruct((_N, 16), _f32),
    )(acc2, b2)


def _build_wext(W, a_s, a_d, H, C, width):
    HC = H * C
    rows_idx = jnp.arange(HC)
    A_s = jnp.zeros((HC, H), _f32).at[rows_idx, rows_idx // C].set(a_s.reshape(-1))
    A_d = jnp.zeros((HC, H), _f32).at[rows_idx, rows_idx // C].set(a_d.reshape(-1))
    wext = jnp.concatenate([W, W @ A_s, W @ A_d], axis=1)
    return jnp.pad(wext, ((0, 0), (0, width - wext.shape[1])))


def kernel(x, edge_index, W1, a_src1, a_dst1, b1, W2, a_src2, a_dst2, b2):
    w1ext = _build_wext(W1, a_src1, a_dst1, 8, 16, 144)
    w2ext = _build_wext(W2, a_src2, a_dst2, 1, 16, 32)
    r8 = jnp.kron(jnp.eye(8, dtype=_f32), jnp.ones((1, 16), _f32))

    loops = jnp.arange(_N, dtype=_i32)
    src = jnp.concatenate([edge_index[0].astype(_i32), loops])
    dst = jnp.concatenate([edge_index[1].astype(_i32), loops])
    pad_n = _E_PAD - _E_TOT
    src = jnp.pad(src, (0, pad_n), constant_values=_N).reshape(_NW, _NCHUNK, _K)
    dst = jnp.pad(dst, (0, pad_n), constant_values=_N).reshape(_NW, _NCHUNK, _K)
    eidx = jnp.stack([src, dst], axis=2)

    z144 = jnp.zeros((_NPAD, 144), _f32)
    z32 = jnp.zeros((_NPAD, 32), _f32)

    t1 = _tc_matmul(x, w1ext)
    t1p = jnp.pad(t1, ((0, _NPAD - _N), (0, 0)))
    d1 = jnp.pad(t1[:, 136:144], ((0, _NPAD - _N), (0, 8)))
    acc1 = _edge1(t1p, d1, eidx, z144)

    t2 = _tc_mid(acc1, b1.reshape(1, 128), w2ext, r8)
    t2p = jnp.pad(t2, ((0, _NPAD - _N), (0, 0)))
    d2 = jnp.pad(t2[:, 17:18], ((0, _NPAD - _N), (0, 15)))
    acc2 = _edge2(t2p, d2, eidx, z32)

    return _tc_fin(acc2, b2.reshape(1, 16))


# speedup vs baseline: 102.2027x; 1.0241x over previous
import functools

import numpy as np
import jax
import jax.numpy as jnp
from jax import lax
from jax.experimental import pallas as pl
from jax.experimental.pallas import tpu as pltpu
from jax.experimental.pallas import tpu_sc as plsc

_N = 10000
_F = 128
_NC, _NS = 2, 16
_NW = _NC * _NS
_K = 96
_E_TOT = 320000 + _N
_NCHUNK = 108
_E_PAD = _NW * _NCHUNK * _K
_RPT = 632
_NPAD = _NS * _RPT

_f32 = jnp.float32
_i32 = jnp.int32


def _make_edge_kernel(H, C):
    HC = H * C
    WS = HC + 16
    mesh = plsc.VectorSubcoreMesh(core_axis_name="c", subcore_axis_name="s")

    def body(tsrc, tdst, eidx_h, out_h,
             cidx0, cidx1, rows0, rows1, arows0, arows1, acc,
             sem_i0, sem_i1, sem_s0, sem_s1, sem_a0, sem_a1):
        cid = lax.axis_index("c")
        sid = lax.axis_index("s")
        g = cid * _NS + sid
        iota = lax.iota(_i32, 16)

        zv = iota.astype(_f32) * 0.0

        def z_body(r, c):
            for col in range(WS // 16):
                rows0[r, pl.ds(col * 16, 16)] = zv
            return c
        lax.fori_loop(0, _K, z_body, 0)
        nfull, nrem = _RPT // _K, _RPT % _K
        for b in range(nfull):
            pltpu.sync_copy(rows0, acc.at[pl.ds(sid * _RPT + b * _K, _K)])
        if nrem:
            pltpu.sync_copy(rows0.at[pl.ds(0, nrem)],
                            acc.at[pl.ds(sid * _RPT + nfull * _K, nrem)])
        plsc.subcore_barrier()

        bufs = ((cidx0, rows0, arows0, sem_i0, sem_s0, sem_a0),
                (cidx1, rows1, arows1, sem_i1, sem_s1, sem_a1))

        def start_idx(j, b):
            cidx, _, _, si, _, _ = bufs[b]
            pltpu.async_copy(eidx_h.at[g, j], cidx, si)

        def wait_idx(j, b):
            cidx, _, _, si, _, _ = bufs[b]
            pltpu.make_async_copy(eidx_h.at[g, j], cidx, si).wait()

        def start_gather(b):
            cidx, rows, arows, _, ss, sa = bufs[b]
            pltpu.async_copy(tsrc.at[cidx.at[0]], rows, ss)
            pltpu.async_copy(tdst.at[cidx.at[1]], arows, sa)

        def wait_gather(b):
            cidx, rows, arows, _, ss, sa = bufs[b]
            pltpu.make_async_copy(tsrc.at[cidx.at[0]], rows, ss).wait()
            pltpu.make_async_copy(tdst.at[cidx.at[1]], arows, sa).wait()

        def compute_scatter(b):
            cidx, rows, arows, _, _, _ = bufs[b]

            @plsc.parallel_loop(0, _K, 1, unroll=4)
            def e_body(e):
                av = rows[e, pl.ds(HC, 16)]
                bv = arows[e, pl.ds(0, 16)]
                a = av + bv
                a = jnp.maximum(a, 0.2 * a)
                ex = jnp.exp(a)
                ex = jnp.where(iota < H, ex, 0.0)
                rows[e, pl.ds(HC, 16)] = ex
                for h in range(H):
                    scv = ex.at[iota * 0 + h].get(mode="promise_in_bounds")
                    off = h * C
                    rows[e, pl.ds(off, 16)] = rows[e, pl.ds(off, 16)] * scv

            pltpu.sync_copy(rows, acc.at[cidx.at[1]], add=True)

        pltpu.sync_copy(eidx_h.at[g, 0], cidx0)
        start_gather(0)
        start_idx(1, 1)

        def step(j, b):
            jn = jnp.minimum(j + 1, _NCHUNK - 1)
            wait_idx(jn, 1 - b)
            start_gather(1 - b)
            wait_gather(b)
            compute_scatter(b)
            start_idx(jnp.minimum(j + 2, _NCHUNK - 1), b)

        def loop_body(jj, c):
            step(jj * 2, 0)
            step(jj * 2 + 1, 1)
            return c

        lax.fori_loop(0, _NCHUNK // 2, loop_body, 0)
        wait_gather(0)
        wait_idx(_NCHUNK - 1, 1)

        plsc.subcore_barrier()
        pltpu.sync_copy(acc.at[pl.ds(sid * _RPT, _RPT)],
                        out_h.at[cid, pl.ds(sid * _RPT, _RPT)])

    return pl.kernel(
        body,
        out_type=jax.ShapeDtypeStruct((_NC, _NPAD, WS), _f32),
        mesh=mesh,
        scratch_types=[
            pltpu.VMEM((2, _K), _i32),
            pltpu.VMEM((2, _K), _i32),
            pltpu.VMEM((_K, WS), _f32),
            pltpu.VMEM((_K, WS), _f32),
            pltpu.VMEM((_K, 16), _f32),
            pltpu.VMEM((_K, 16), _f32),
            pltpu.VMEM_SHARED((_NPAD, WS), _f32),
            pltpu.SemaphoreType.DMA,
            pltpu.SemaphoreType.DMA,
            pltpu.SemaphoreType.DMA,
            pltpu.SemaphoreType.DMA,
            pltpu.SemaphoreType.DMA,
            pltpu.SemaphoreType.DMA,
        ],
        compiler_params=pltpu.CompilerParams(use_tc_tiling_on_sc=False),
    )


_edge1 = _make_edge_kernel(8, 16)
_edge2 = _make_edge_kernel(1, 16)


_BM = 632
_BMF = 1000


def _mm_body(x_ref, w_ref, o1_ref, o2_ref):
    w = w_ref[...]
    t = jnp.dot(x_ref[...], w, preferred_element_type=_f32)
    n1 = o1_ref.shape[1]
    o1_ref[...] = t[:, :n1]
    o2_ref[...] = t[:, n1:]


def _tc_matmul2(x, w, n1):
    m, k = x.shape
    n = w.shape[1]
    return pl.pallas_call(
        _mm_body,
        grid=(m // _BM,),
        in_specs=[pl.BlockSpec((_BM, k), lambda i: (i, 0)),
                  pl.BlockSpec((k, n), lambda i: (0, 0))],
        out_specs=[pl.BlockSpec((_BM, n1), lambda i: (i, 0)),
                   pl.BlockSpec((_BM, n - n1), lambda i: (i, 0))],
        out_shape=[jax.ShapeDtypeStruct((m, n1), _f32),
                   jax.ShapeDtypeStruct((m, n - n1), _f32)],
    )(x, w)


def _mid_body(acc_ref, b_ref, w_ref, r_ref, o1_ref, o2_ref):
    s = acc_ref[0] + acc_ref[1]
    num = s[:, :128]
    den = jnp.dot(s[:, 128:136], r_ref[...], preferred_element_type=_f32)
    gv = num / den + b_ref[...]
    gv = jnp.where(gv > 0, gv, jnp.exp(gv) - 1.0)
    t = jnp.dot(gv, w_ref[...], preferred_element_type=_f32)
    o1_ref[...] = t[:, :32]
    o2_ref[...] = t[:, 32:]


def _tc_mid(acc1, b1, w2ext, r8):
    return pl.pallas_call(
        _mid_body,
        grid=(_NPAD // _BM,),
        in_specs=[pl.BlockSpec((_NC, _BM, 144), lambda i: (0, i, 0)),
                  pl.BlockSpec((1, 128), lambda i: (0, 0)),
                  pl.BlockSpec((128, 48), lambda i: (0, 0)),
                  pl.BlockSpec((8, 128), lambda i: (0, 0))],
        out_specs=[pl.BlockSpec((_BM, 32), lambda i: (i, 0)),
                   pl.BlockSpec((_BM, 16), lambda i: (i, 0))],
        out_shape=[jax.ShapeDtypeStruct((_NPAD, 32), _f32),
                   jax.ShapeDtypeStruct((_NPAD, 16), _f32)],
    )(acc1, b1, w2ext, r8)


def _fin_body(acc_ref, b_ref, o_ref):
    s = acc_ref[0] + acc_ref[1]
    num = s[:, :16]
    den = jnp.broadcast_to(s[:, 16:17], (_BMF, 16))
    o = num / den + b_ref[...]
    o_ref[...] = jnp.where(o > 0, o, jnp.exp(o) - 1.0)


def _tc_fin(acc2, b2):
    return pl.pallas_call(
        _fin_body,
        grid=(_N // _BMF,),
        in_specs=[pl.BlockSpec((_NC, _BMF, 32), lambda i: (0, i, 0)),
                  pl.BlockSpec((1, 16), lambda i: (0, 0))],
        out_specs=pl.BlockSpec((_BMF, 16), lambda i: (i, 0)),
        out_shape=jax.ShapeDtypeStruct((_N, 16), _f32),
    )(acc2, b2)


def _build_wext(W, a_s, a_d, H, C, width):
    HC = H * C
    rows_idx = jnp.arange(HC)
    A_s = jnp.zeros((HC, H), _f32).at[rows_idx, rows_idx // C].set(a_s.reshape(-1))
    A_d = jnp.zeros((HC, H), _f32).at[rows_idx, rows_idx // C].set(a_d.reshape(-1))
    wext = jnp.concatenate([W, W @ A_s, W @ A_d], axis=1)
    return jnp.pad(wext, ((0, 0), (0, width - wext.shape[1])))


def kernel(x, edge_index, W1, a_src1, a_dst1, b1, W2, a_src2, a_dst2, b2):
    w1ext = _build_wext(W1, a_src1, a_dst1, 8, 16, 144)
    A1d = jnp.zeros((128, 8), _f32).at[
        jnp.arange(128), jnp.arange(128) // 16].set(a_dst1.reshape(-1))
    w1full = jnp.concatenate(
        [w1ext, W1 @ A1d, jnp.zeros((128, 8), _f32)], axis=1)
    w2ext = _build_wext(W2, a_src2, a_dst2, 1, 16, 32)
    w2full = jnp.concatenate(
        [w2ext, W2 @ a_dst2.T, jnp.zeros((128, 15), _f32)], axis=1)
    r8 = jnp.kron(jnp.eye(8, dtype=_f32), jnp.ones((1, 16), _f32))

    loops = jnp.arange(_N, dtype=_i32)
    src = jnp.concatenate([edge_index[0].astype(_i32), loops])
    dst = jnp.concatenate([edge_index[1].astype(_i32), loops])
    pad_n = _E_PAD - _E_TOT
    src = jnp.pad(src, (0, pad_n), constant_values=_N).reshape(_NW, _NCHUNK, _K)
    dst = jnp.pad(dst, (0, pad_n), constant_values=_N).reshape(_NW, _NCHUNK, _K)
    eidx = jnp.stack([src, dst], axis=2)

    xp = jnp.pad(x, ((0, _NPAD - _N), (0, 0)))

    t1p, d1 = _tc_matmul2(xp, w1full, 144)
    acc1 = _edge1(t1p, d1, eidx)

    t2p, d2 = _tc_mid(acc1, b1.reshape(1, 128), w2full, r8)
    acc2 = _edge2(t2p, d2, eidx)

    return _tc_fin(acc2, b2.reshape(1, 16))
